# baseline (speedup 1.0000x reference)
import functools

import jax
import jax.numpy as jnp
from jax import lax
from jax.experimental import pallas as pl
from jax.experimental.pallas import tpu as pltpu
from jax.experimental.pallas import tpu_sc as plsc

B = 2048
D = 768
E = 8
H = 1024
T = 256
NPAD = 2 * B + E * T
NBLK = NPAD // T
NBLKPAD = 32
NC = 2
NS = 16
NW = NC * NS
CB = B // NW
LOG2T = 8

@functools.cache
def _mesh():
    return plsc.VectorSubcoreMesh(
        core_axis_name="c", subcore_axis_name="s",
        num_cores=NC, num_subcores=NS)


def _gate_kernel(x_ref, gw_ref, gb_ref,
                 scores_ref, a1_ref, a2_ref, w1_ref, w2_ref):
    logits = jnp.dot(x_ref[...], gw_ref[...],
                     preferred_element_type=jnp.float32) + gb_ref[...]
    m = jnp.max(logits, axis=-1, keepdims=True)
    ex = jnp.exp(logits - m)
    p = ex / jnp.sum(ex, axis=-1, keepdims=True)
    scores_ref[...] = p
    iota = lax.broadcasted_iota(jnp.int32, p.shape, 1)
    m1 = jnp.max(p, axis=-1, keepdims=True)
    a1 = jnp.min(jnp.where(p == m1, iota, E), axis=-1, keepdims=True)
    p2 = jnp.where(iota == a1, -jnp.inf, p)
    m2 = jnp.max(p2, axis=-1, keepdims=True)
    a2 = jnp.min(jnp.where(p2 == m2, iota, E), axis=-1, keepdims=True)
    a1_ref[...] = a1
    a2_ref[...] = a2
    w1_ref[...] = m1
    w2_ref[...] = m2


def _gate(x, gate_w, gate_b):
    return pl.pallas_call(
        _gate_kernel,
        grid=(1,),
        in_specs=[
            pl.BlockSpec((B, D), lambda i: (0, 0)),
            pl.BlockSpec((D, E), lambda i: (0, 0)),
            pl.BlockSpec((1, E), lambda i: (0, 0)),
        ],
        out_specs=[
            pl.BlockSpec((B, E), lambda i: (0, 0)),
            pl.BlockSpec((B, 1), lambda i: (0, 0)),
            pl.BlockSpec((B, 1), lambda i: (0, 0)),
            pl.BlockSpec((B, 1), lambda i: (0, 0)),
            pl.BlockSpec((B, 1), lambda i: (0, 0)),
        ],
        out_shape=[
            jax.ShapeDtypeStruct((B, E), jnp.float32),
            jax.ShapeDtypeStruct((B, 1), jnp.int32),
            jax.ShapeDtypeStruct((B, 1), jnp.int32),
            jax.ShapeDtypeStruct((B, 1), jnp.float32),
            jax.ShapeDtypeStruct((B, 1), jnp.float32),
        ],
    )(x, gate_w, gate_b.reshape(1, E))


@functools.cache
def _dispatch_call():
    return pl.kernel(
        _dispatch,
        out_type=[
            jax.ShapeDtypeStruct((NPAD, D), jnp.float32),
            jax.ShapeDtypeStruct((B,), jnp.int32),
            jax.ShapeDtypeStruct((B,), jnp.int32),
            jax.ShapeDtypeStruct((NBLKPAD,), jnp.int32),
        ],
        mesh=_mesh(),
        scratch_types=[
            pltpu.VMEM((2 * B,), jnp.int32),
            pltpu.VMEM((CB, D), jnp.float32),
            pltpu.VMEM((CB,), jnp.int32),
            pltpu.VMEM((CB,), jnp.int32),
            pltpu.VMEM((NBLKPAD,), jnp.int32),
            pltpu.VMEM((16,), jnp.int32),
            pltpu.SemaphoreType.DMA,
            pltpu.SemaphoreType.DMA,
            pltpu.SemaphoreType.DMA,
        ],
        compiler_params=pltpu.CompilerParams(needs_layout_passes=False),
    )


def _dispatch(ecat_hbm, x_hbm,
              xs_hbm, pos1_hbm, pos2_hbm, bexp_hbm,
              eall_v, rows_v, pos1_v, pos2_v, bexp_v, tmp16_v,
              sem1, sem2, sem3):
    wid = lax.axis_index("s") * NC + lax.axis_index("c")
    base = wid * CB
    lane = lax.broadcasted_iota(jnp.int32, (16,), 0)
    zero16 = jnp.zeros((16,), jnp.int32)

    cpr = pltpu.async_copy(x_hbm.at[pl.ds(base, CB)], rows_v, sem3)
    pltpu.sync_copy(ecat_hbm, eall_v)

    def _cumsum16(v):
        for s in (1, 2, 4, 8):
            tmp16_v[...] = v
            idx = jnp.maximum(lane - s, 0)
            shifted = plsc.load_gather(tmp16_v, [idx])
            v = v + jnp.where(lane >= s, shifted, 0)
        return v

    nch1 = B // 16
    my1 = wid * (CB // 16)
    my2 = nch1 + wid * (CB // 16)

    def seg_body(j, carry):
        chunk = eall_v[pl.ds(j * 16, 16)]
        return tuple(carry[e] + jnp.where(chunk == e, 1, 0) for e in range(E))

    zeros = (zero16,) * E
    seg_a = lax.fori_loop(0, my1, seg_body, zeros)
    seg_b = lax.fori_loop(my1, my2, seg_body, zeros)
    seg_c = lax.fori_loop(my2, 2 * nch1, seg_body, zeros)
    tot = zero16
    pre1 = zero16
    pre2 = zero16
    for e in range(E):
        sel = jnp.where(lane == e, 1, 0)
        ab = seg_a[e] + seg_b[e]
        pre1 = pre1 + sel * _cumsum16(seg_a[e])[15]
        pre2 = pre2 + sel * _cumsum16(ab)[15]
        tot = tot + sel * _cumsum16(ab + seg_c[e])[15]

    al = ((tot + (T - 1)) >> LOG2T) << LOG2T
    al = jnp.where(lane < E, al, 0)
    gs_incl = _cumsum16(al)
    gstart = gs_incl - al
    start1 = gstart + pre1
    start2 = gstart + pre2

    for sec, pos_v in ((0, pos1_v), (1, pos2_v)):
        svec = start1 if sec == 0 else start2
        run = [svec[e] for e in range(E)]
        sec_off = sec * B
        for c in range(CB // 16):
            off = sec_off + base + c * 16
            chunk = eall_v[pl.ds(off, 16)]
            pos = zero16
            for e in range(E):
                mi = jnp.where(chunk == e, 1, 0)
                cs = _cumsum16(mi)
                pos = pos + mi * (run[e] + cs - 1)
                run[e] = run[e] + cs[15]
            pos_v[pl.ds(c * 16, 16)] = pos

    pltpu.sync_copy(pos1_v, pos1_hbm.at[pl.ds(base, CB)])
    pltpu.sync_copy(pos2_v, pos2_hbm.at[pl.ds(base, CB)])

    cpr.wait()
    cp1 = pltpu.async_copy(rows_v, xs_hbm.at[pos1_v], sem1)
    cp2 = pltpu.async_copy(rows_v, xs_hbm.at[pos2_v], sem2)
    cp1.wait()
    cp2.wait()

    @pl.when(wid == 0)
    def _bexp():
        used_end = (gstart[E - 1] + al[E - 1]) >> LOG2T
        for j in range(NBLKPAD // 16):
            blk = lane + j * 16
            acc = zero16
            for e in range(1, E):
                acc = acc + jnp.where(blk >= (gstart[e] >> LOG2T), 1, 0)
            bev = jnp.where(blk < used_end, acc, -1)
            bev = jnp.where(blk == NBLK, used_end, bev)
            bexp_v[pl.ds(j * 16, 16)] = bev
        pltpu.sync_copy(bexp_v, bexp_hbm)


def _ffn_kernel(be_ref, xs_ref, W1_ref, b1_ref, W2_ref, b2_ref, ys_ref):
    xb = xs_ref[...].astype(jnp.bfloat16)
    h = jnp.maximum(
        jnp.dot(xb, W1_ref[0].astype(jnp.bfloat16),
                preferred_element_type=jnp.float32) + b1_ref[0], 0.0)
    ys_ref[...] = jnp.dot(h.astype(jnp.bfloat16),
                          W2_ref[0].astype(jnp.bfloat16),
                          preferred_element_type=jnp.float32) + b2_ref[0]


def _ffn(bexp, xs, W1, b1r, W2, b2r):
    def xidx(i, be):
        return jnp.minimum(i, be[NBLK] - 1)

    def widx(i, be):
        return be[jnp.minimum(i, be[NBLK] - 1)]

    grid_spec = pltpu.PrefetchScalarGridSpec(
        num_scalar_prefetch=1,
        grid=(NBLK,),
        in_specs=[
            pl.BlockSpec((T, D), lambda i, be: (xidx(i, be), 0)),
            pl.BlockSpec((1, D, H), lambda i, be: (widx(i, be), 0, 0)),
            pl.BlockSpec((1, 1, H), lambda i, be: (widx(i, be), 0, 0)),
            pl.BlockSpec((1, H, D), lambda i, be: (widx(i, be), 0, 0)),
            pl.BlockSpec((1, 1, D), lambda i, be: (widx(i, be), 0, 0)),
        ],
        out_specs=pl.BlockSpec((T, D), lambda i, be: (xidx(i, be), 0)),
    )
    return pl.pallas_call(
        _ffn_kernel,
        grid_spec=grid_spec,
        out_shape=jax.ShapeDtypeStruct((NPAD, D), jnp.float32),
    )(bexp, xs, W1, b1r, W2, b2r)


@functools.cache
def _combine_call():
    return pl.kernel(
        _combine,
        out_type=jax.ShapeDtypeStruct((B, D), jnp.float32),
        mesh=_mesh(),
        scratch_types=[
            pltpu.VMEM((CB,), jnp.int32),
            pltpu.VMEM((CB,), jnp.int32),
            pltpu.VMEM((CB,), jnp.float32),
            pltpu.VMEM((CB,), jnp.float32),
            pltpu.VMEM((CB, D), jnp.float32),
            pltpu.VMEM((CB, D), jnp.float32),
            pltpu.SemaphoreType.DMA,
            pltpu.SemaphoreType.DMA,
        ],
        compiler_params=pltpu.CompilerParams(needs_layout_passes=False),
    )


def _combine(ys_hbm, pos1_hbm, pos2_hbm, w1_hbm, w2_hbm,
             out_hbm,
             p1_v, p2_v, wa_v, wb_v, r1_v, r2_v, sem1, sem2):
    wid = lax.axis_index("s") * NC + lax.axis_index("c")
    base = wid * CB
    pltpu.sync_copy(pos1_hbm.at[pl.ds(base, CB)], p1_v)
    pltpu.sync_copy(pos2_hbm.at[pl.ds(base, CB)], p2_v)
    pltpu.sync_copy(w1_hbm.at[pl.ds(base, CB)], wa_v)
    pltpu.sync_copy(w2_hbm.at[pl.ds(base, CB)], wb_v)
    cp1 = pltpu.async_copy(ys_hbm.at[p1_v], r1_v, sem1)
    cp2 = pltpu.async_copy(ys_hbm.at[p2_v], r2_v, sem2)
    cp1.wait()
    cp2.wait()

    def tbody(t, carry):
        idx = jnp.full((16,), t, jnp.int32)
        wa = plsc.load_gather(wa_v, [idx])
        wb = plsc.load_gather(wb_v, [idx])
        for c in range(D // 16):
            sl = pl.ds(c * 16, 16)
            r1_v[t, sl] = r1_v[t, sl] * wa + r2_v[t, sl] * wb
        return carry

    lax.fori_loop(0, CB, tbody, 0)
    pltpu.sync_copy(r1_v, out_hbm.at[pl.ds(base, CB)])


def kernel(x, gate_w, gate_b, W1, b1, W2, b2):
    scores, a1, a2, w1c, w2c = _gate(x, gate_w, gate_b)
    ecat = jnp.concatenate([a1.reshape(B), a2.reshape(B)])
    xs, pos1, pos2, bexp = _dispatch_call()(ecat, x)
    ys = _ffn(bexp, xs, W1, b1.reshape(E, 1, H), W2, b2.reshape(E, 1, D))
    out = _combine_call()(ys, pos1, pos2, w1c.reshape(B), w2c.reshape(B))
    return (out, lax.stop_gradient(scores))


# speedup vs baseline: 1.1375x; 1.1375x over previous
import functools

import jax
import jax.numpy as jnp
from jax import lax
from jax.experimental import pallas as pl
from jax.experimental.pallas import tpu as pltpu
from jax.experimental.pallas import tpu_sc as plsc

B = 2048
D = 768
E = 8
H = 1024
T = 512
NPAD = 2 * B + E * T
NBLK = NPAD // T
NBLKPAD = 32
NC = 2
NS = 16
NW = NC * NS
CB = B // NW
LOG2T = 9

@functools.cache
def _mesh():
    return plsc.VectorSubcoreMesh(
        core_axis_name="c", subcore_axis_name="s",
        num_cores=NC, num_subcores=NS)


def _gate_kernel(x_ref, gw_ref, gb_ref,
                 scores_ref, a1_ref, a2_ref, w1_ref, w2_ref):
    logits = jnp.dot(x_ref[...], gw_ref[...],
                     preferred_element_type=jnp.float32) + gb_ref[...]
    m = jnp.max(logits, axis=-1, keepdims=True)
    ex = jnp.exp(logits - m)
    p = ex / jnp.sum(ex, axis=-1, keepdims=True)
    scores_ref[...] = p
    iota = lax.broadcasted_iota(jnp.int32, p.shape, 1)
    m1 = jnp.max(p, axis=-1, keepdims=True)
    a1 = jnp.min(jnp.where(p == m1, iota, E), axis=-1, keepdims=True)
    p2 = jnp.where(iota == a1, -jnp.inf, p)
    m2 = jnp.max(p2, axis=-1, keepdims=True)
    a2 = jnp.min(jnp.where(p2 == m2, iota, E), axis=-1, keepdims=True)
    a1_ref[...] = a1
    a2_ref[...] = a2
    w1_ref[...] = m1
    w2_ref[...] = m2


def _gate(x, gate_w, gate_b):
    return pl.pallas_call(
        _gate_kernel,
        grid=(1,),
        in_specs=[
            pl.BlockSpec((B, D), lambda i: (0, 0)),
            pl.BlockSpec((D, E), lambda i: (0, 0)),
            pl.BlockSpec((1, E), lambda i: (0, 0)),
        ],
        out_specs=[
            pl.BlockSpec((B, E), lambda i: (0, 0)),
            pl.BlockSpec((B, 1), lambda i: (0, 0)),
            pl.BlockSpec((B, 1), lambda i: (0, 0)),
            pl.BlockSpec((B, 1), lambda i: (0, 0)),
            pl.BlockSpec((B, 1), lambda i: (0, 0)),
        ],
        out_shape=[
            jax.ShapeDtypeStruct((B, E), jnp.float32),
            jax.ShapeDtypeStruct((B, 1), jnp.int32),
            jax.ShapeDtypeStruct((B, 1), jnp.int32),
            jax.ShapeDtypeStruct((B, 1), jnp.float32),
            jax.ShapeDtypeStruct((B, 1), jnp.float32),
        ],
    )(x, gate_w, gate_b.reshape(1, E))


@functools.cache
def _dispatch_call():
    return pl.kernel(
        _dispatch,
        out_type=[
            jax.ShapeDtypeStruct((NPAD, D), jnp.float32),
            jax.ShapeDtypeStruct((B,), jnp.int32),
            jax.ShapeDtypeStruct((B,), jnp.int32),
            jax.ShapeDtypeStruct((NBLKPAD,), jnp.int32),
        ],
        mesh=_mesh(),
        scratch_types=[
            pltpu.VMEM((2 * B,), jnp.int32),
            pltpu.VMEM((CB, D), jnp.float32),
            pltpu.VMEM((CB,), jnp.int32),
            pltpu.VMEM((CB,), jnp.int32),
            pltpu.VMEM((NBLKPAD,), jnp.int32),
            pltpu.VMEM((16,), jnp.int32),
            pltpu.SemaphoreType.DMA,
            pltpu.SemaphoreType.DMA,
            pltpu.SemaphoreType.DMA,
        ],
        compiler_params=pltpu.CompilerParams(needs_layout_passes=False),
    )


def _dispatch(ecat_hbm, x_hbm,
              xs_hbm, pos1_hbm, pos2_hbm, bexp_hbm,
              eall_v, rows_v, pos1_v, pos2_v, bexp_v, tmp16_v,
              sem1, sem2, sem3):
    wid = lax.axis_index("s") * NC + lax.axis_index("c")
    base = wid * CB
    lane = lax.broadcasted_iota(jnp.int32, (16,), 0)
    zero16 = jnp.zeros((16,), jnp.int32)

    cpr = pltpu.async_copy(x_hbm.at[pl.ds(base, CB)], rows_v, sem3)
    pltpu.sync_copy(ecat_hbm, eall_v)

    def _cumsum16(v):
        for s in (1, 2, 4, 8):
            tmp16_v[...] = v
            idx = jnp.maximum(lane - s, 0)
            shifted = plsc.load_gather(tmp16_v, [idx])
            v = v + jnp.where(lane >= s, shifted, 0)
        return v

    nch1 = B // 16
    my1 = wid * (CB // 16)
    my2 = nch1 + wid * (CB // 16)

    def seg_body(j, carry):
        chunk = eall_v[pl.ds(j * 16, 16)]
        return tuple(carry[e] + jnp.where(chunk == e, 1, 0) for e in range(E))

    zeros = (zero16,) * E
    seg_a = lax.fori_loop(0, my1, seg_body, zeros)
    seg_b = lax.fori_loop(my1, my2, seg_body, zeros)
    seg_c = lax.fori_loop(my2, 2 * nch1, seg_body, zeros)
    tot = zero16
    pre1 = zero16
    pre2 = zero16
    for e in range(E):
        sel = jnp.where(lane == e, 1, 0)
        ab = seg_a[e] + seg_b[e]
        pre1 = pre1 + sel * _cumsum16(seg_a[e])[15]
        pre2 = pre2 + sel * _cumsum16(ab)[15]
        tot = tot + sel * _cumsum16(ab + seg_c[e])[15]

    al = ((tot + (T - 1)) >> LOG2T) << LOG2T
    al = jnp.where(lane < E, al, 0)
    gs_incl = _cumsum16(al)
    gstart = gs_incl - al
    start1 = gstart + pre1
    start2 = gstart + pre2

    for sec, pos_v in ((0, pos1_v), (1, pos2_v)):
        svec = start1 if sec == 0 else start2
        run = [svec[e] for e in range(E)]
        sec_off = sec * B
        for c in range(CB // 16):
            off = sec_off + base + c * 16
            chunk = eall_v[pl.ds(off, 16)]
            pos = zero16
            for e in range(E):
                mi = jnp.where(chunk == e, 1, 0)
                cs = _cumsum16(mi)
                pos = pos + mi * (run[e] + cs - 1)
                run[e] = run[e] + cs[15]
            pos_v[pl.ds(c * 16, 16)] = pos

    pltpu.sync_copy(pos1_v, pos1_hbm.at[pl.ds(base, CB)])
    pltpu.sync_copy(pos2_v, pos2_hbm.at[pl.ds(base, CB)])

    cpr.wait()
    cp1 = pltpu.async_copy(rows_v, xs_hbm.at[pos1_v], sem1)
    cp2 = pltpu.async_copy(rows_v, xs_hbm.at[pos2_v], sem2)
    cp1.wait()
    cp2.wait()

    @pl.when(wid == 0)
    def _bexp():
        used_end = (gstart[E - 1] + al[E - 1]) >> LOG2T
        for j in range(NBLKPAD // 16):
            blk = lane + j * 16
            acc = zero16
            for e in range(1, E):
                acc = acc + jnp.where(blk >= (gstart[e] >> LOG2T), 1, 0)
            bev = jnp.where(blk < used_end, acc, -1)
            bev = jnp.where(blk == NBLK, used_end, bev)
            bexp_v[pl.ds(j * 16, 16)] = bev
        pltpu.sync_copy(bexp_v, bexp_hbm)


def _ffn_kernel(be_ref, xs_ref, W1_ref, b1_ref, W2_ref, b2_ref, ys_ref):
    i = pl.program_id(0)

    @pl.when(be_ref[i] >= 0)
    def _():
        xb = xs_ref[...].astype(jnp.bfloat16)
        h = jnp.maximum(
            jnp.dot(xb, W1_ref[0].astype(jnp.bfloat16),
                    preferred_element_type=jnp.float32) + b1_ref[0], 0.0)
        ys_ref[...] = jnp.dot(h.astype(jnp.bfloat16),
                              W2_ref[0].astype(jnp.bfloat16),
                              preferred_element_type=jnp.float32) + b2_ref[0]


def _ffn(bexp, xs, W1, b1r, W2, b2r):
    def xidx(i, be):
        return jnp.minimum(i, be[NBLK] - 1)

    def widx(i, be):
        return be[jnp.minimum(i, be[NBLK] - 1)]

    grid_spec = pltpu.PrefetchScalarGridSpec(
        num_scalar_prefetch=1,
        grid=(NBLK,),
        in_specs=[
            pl.BlockSpec((T, D), lambda i, be: (xidx(i, be), 0)),
            pl.BlockSpec((1, D, H), lambda i, be: (widx(i, be), 0, 0)),
            pl.BlockSpec((1, 1, H), lambda i, be: (widx(i, be), 0, 0)),
            pl.BlockSpec((1, H, D), lambda i, be: (widx(i, be), 0, 0)),
            pl.BlockSpec((1, 1, D), lambda i, be: (widx(i, be), 0, 0)),
        ],
        out_specs=pl.BlockSpec((T, D), lambda i, be: (xidx(i, be), 0)),
    )
    return pl.pallas_call(
        _ffn_kernel,
        grid_spec=grid_spec,
        out_shape=jax.ShapeDtypeStruct((NPAD, D), jnp.float32),
    )(bexp, xs, W1, b1r, W2, b2r)


@functools.cache
def _combine_call():
    return pl.kernel(
        _combine,
        out_type=jax.ShapeDtypeStruct((B, D), jnp.float32),
        mesh=_mesh(),
        scratch_types=[
            pltpu.VMEM((CB,), jnp.int32),
            pltpu.VMEM((CB,), jnp.int32),
            pltpu.VMEM((CB,), jnp.float32),
            pltpu.VMEM((CB,), jnp.float32),
            pltpu.VMEM((CB, D), jnp.float32),
            pltpu.VMEM((CB, D), jnp.float32),
            pltpu.SemaphoreType.DMA,
            pltpu.SemaphoreType.DMA,
        ],
        compiler_params=pltpu.CompilerParams(needs_layout_passes=False),
    )


def _combine(ys_hbm, pos1_hbm, pos2_hbm, w1_hbm, w2_hbm,
             out_hbm,
             p1_v, p2_v, wa_v, wb_v, r1_v, r2_v, sem1, sem2):
    wid = lax.axis_index("s") * NC + lax.axis_index("c")
    base = wid * CB
    pltpu.sync_copy(pos1_hbm.at[pl.ds(base, CB)], p1_v)
    pltpu.sync_copy(pos2_hbm.at[pl.ds(base, CB)], p2_v)
    pltpu.sync_copy(w1_hbm.at[pl.ds(base, CB)], wa_v)
    pltpu.sync_copy(w2_hbm.at[pl.ds(base, CB)], wb_v)
    cp1 = pltpu.async_copy(ys_hbm.at[p1_v], r1_v, sem1)
    cp2 = pltpu.async_copy(ys_hbm.at[p2_v], r2_v, sem2)
    cp1.wait()
    cp2.wait()

    def tbody(t, carry):
        idx = jnp.full((16,), t, jnp.int32)
        wa = plsc.load_gather(wa_v, [idx])
        wb = plsc.load_gather(wb_v, [idx])
        for c in range(D // 16):
            sl = pl.ds(c * 16, 16)
            r1_v[t, sl] = r1_v[t, sl] * wa + r2_v[t, sl] * wb
        return carry

    lax.fori_loop(0, CB, tbody, 0)
    pltpu.sync_copy(r1_v, out_hbm.at[pl.ds(base, CB)])


def kernel(x, gate_w, gate_b, W1, b1, W2, b2):
    scores, a1, a2, w1c, w2c = _gate(x, gate_w, gate_b)
    ecat = jnp.concatenate([a1.reshape(B), a2.reshape(B)])
    xs, pos1, pos2, bexp = _dispatch_call()(ecat, x)
    ys = _ffn(bexp, xs, W1, b1.reshape(E, 1, H), W2, b2.reshape(E, 1, D))
    out = _combine_call()(ys, pos1, pos2, w1c.reshape(B), w2c.reshape(B))
    return (out, lax.stop_gradient(scores))


# speedup vs baseline: 1.1382x; 1.0006x over previous
import functools

import jax
import jax.numpy as jnp
from jax import lax
from jax.experimental import pallas as pl
from jax.experimental.pallas import tpu as pltpu
from jax.experimental.pallas import tpu_sc as plsc

B = 2048
D = 768
E = 8
H = 1024
T = 512
NPAD = 2 * B + E * T
NBLK = NPAD // T
NBLKPAD = 32
NC = 2
NS = 16
NW = NC * NS
CB = B // NW
LOG2T = 9

@functools.cache
def _mesh():
    return plsc.VectorSubcoreMesh(
        core_axis_name="c", subcore_axis_name="s",
        num_cores=NC, num_subcores=NS)


def _gate_kernel(x_ref, gw_ref, gb_ref,
                 scores_ref, a1_ref, a2_ref, w1_ref, w2_ref):
    logits = jnp.dot(x_ref[...], gw_ref[...],
                     preferred_element_type=jnp.float32) + gb_ref[...]
    m = jnp.max(logits, axis=-1, keepdims=True)
    ex = jnp.exp(logits - m)
    p = ex / jnp.sum(ex, axis=-1, keepdims=True)
    scores_ref[...] = p
    iota = lax.broadcasted_iota(jnp.int32, p.shape, 1)
    m1 = jnp.max(p, axis=-1, keepdims=True)
    a1 = jnp.min(jnp.where(p == m1, iota, E), axis=-1, keepdims=True)
    p2 = jnp.where(iota == a1, -jnp.inf, p)
    m2 = jnp.max(p2, axis=-1, keepdims=True)
    a2 = jnp.min(jnp.where(p2 == m2, iota, E), axis=-1, keepdims=True)
    a1_ref[...] = a1
    a2_ref[...] = a2
    w1_ref[...] = m1
    w2_ref[...] = m2


def _gate(x, gate_w, gate_b):
    return pl.pallas_call(
        _gate_kernel,
        grid=(1,),
        in_specs=[
            pl.BlockSpec((B, D), lambda i: (0, 0)),
            pl.BlockSpec((D, E), lambda i: (0, 0)),
            pl.BlockSpec((1, E), lambda i: (0, 0)),
        ],
        out_specs=[
            pl.BlockSpec((B, E), lambda i: (0, 0)),
            pl.BlockSpec((B, 1), lambda i: (0, 0)),
            pl.BlockSpec((B, 1), lambda i: (0, 0)),
            pl.BlockSpec((B, 1), lambda i: (0, 0)),
            pl.BlockSpec((B, 1), lambda i: (0, 0)),
        ],
        out_shape=[
            jax.ShapeDtypeStruct((B, E), jnp.float32),
            jax.ShapeDtypeStruct((B, 1), jnp.int32),
            jax.ShapeDtypeStruct((B, 1), jnp.int32),
            jax.ShapeDtypeStruct((B, 1), jnp.float32),
            jax.ShapeDtypeStruct((B, 1), jnp.float32),
        ],
    )(x, gate_w, gate_b.reshape(1, E))


@functools.cache
def _dispatch_call():
    return pl.kernel(
        _dispatch,
        out_type=[
            jax.ShapeDtypeStruct((NPAD, D), jnp.float32),
            jax.ShapeDtypeStruct((B,), jnp.int32),
            jax.ShapeDtypeStruct((B,), jnp.int32),
            jax.ShapeDtypeStruct((NBLKPAD,), jnp.int32),
        ],
        mesh=_mesh(),
        scratch_types=[
            pltpu.VMEM((2 * B,), jnp.int32),
            pltpu.VMEM((CB, D), jnp.float32),
            pltpu.VMEM((CB,), jnp.int32),
            pltpu.VMEM((CB,), jnp.int32),
            pltpu.VMEM((NBLKPAD,), jnp.int32),
            pltpu.VMEM((16,), jnp.int32),
            pltpu.SemaphoreType.DMA,
            pltpu.SemaphoreType.DMA,
            pltpu.SemaphoreType.DMA,
        ],
        compiler_params=pltpu.CompilerParams(needs_layout_passes=False),
    )


def _dispatch(ecat_hbm, x_hbm,
              xs_hbm, pos1_hbm, pos2_hbm, bexp_hbm,
              eall_v, rows_v, pos1_v, pos2_v, bexp_v, tmp16_v,
              sem1, sem2, sem3):
    wid = lax.axis_index("s") * NC + lax.axis_index("c")
    base = wid * CB
    lane = lax.broadcasted_iota(jnp.int32, (16,), 0)
    zero16 = jnp.zeros((16,), jnp.int32)

    cpr = pltpu.async_copy(x_hbm.at[pl.ds(base, CB)], rows_v, sem3)
    pltpu.sync_copy(ecat_hbm, eall_v)

    def _cumsum16(v):
        for s in (1, 2, 4, 8):
            tmp16_v[...] = v
            idx = jnp.maximum(lane - s, 0)
            shifted = plsc.load_gather(tmp16_v, [idx])
            v = v + jnp.where(lane >= s, shifted, 0)
        return v

    nch1 = B // 16
    my1 = wid * (CB // 16)
    my2 = nch1 + wid * (CB // 16)

    def seg_body(j, carry):
        chunk = eall_v[pl.ds(j * 16, 16)]
        return tuple(carry[e] + jnp.where(chunk == e, 1, 0) for e in range(E))

    zeros = (zero16,) * E
    seg_a = lax.fori_loop(0, my1, seg_body, zeros)
    seg_b = lax.fori_loop(my1, my2, seg_body, zeros)
    seg_c = lax.fori_loop(my2, 2 * nch1, seg_body, zeros)
    tot = zero16
    pre1 = zero16
    pre2 = zero16
    for e in range(E):
        sel = jnp.where(lane == e, 1, 0)
        ab = seg_a[e] + seg_b[e]
        pre1 = pre1 + sel * _cumsum16(seg_a[e])[15]
        pre2 = pre2 + sel * _cumsum16(ab)[15]
        tot = tot + sel * _cumsum16(ab + seg_c[e])[15]

    al = ((tot + (T - 1)) >> LOG2T) << LOG2T
    al = jnp.where(lane < E, al, 0)
    gs_incl = _cumsum16(al)
    gstart = gs_incl - al
    start1 = gstart + pre1
    start2 = gstart + pre2

    for sec, pos_v in ((0, pos1_v), (1, pos2_v)):
        svec = start1 if sec == 0 else start2
        run = [svec[e] for e in range(E)]
        sec_off = sec * B
        for c in range(CB // 16):
            off = sec_off + base + c * 16
            chunk = eall_v[pl.ds(off, 16)]
            pos = zero16
            for e in range(E):
                mi = jnp.where(chunk == e, 1, 0)
                cs = _cumsum16(mi)
                pos = pos + mi * (run[e] + cs - 1)
                run[e] = run[e] + cs[15]
            pos_v[pl.ds(c * 16, 16)] = pos

    pltpu.sync_copy(pos1_v, pos1_hbm.at[pl.ds(base, CB)])
    pltpu.sync_copy(pos2_v, pos2_hbm.at[pl.ds(base, CB)])

    cpr.wait()
    cp1 = pltpu.async_copy(rows_v, xs_hbm.at[pos1_v], sem1)
    cp2 = pltpu.async_copy(rows_v, xs_hbm.at[pos2_v], sem2)
    cp1.wait()
    cp2.wait()

    @pl.when(wid == 0)
    def _bexp():
        used_end = (gstart[E - 1] + al[E - 1]) >> LOG2T
        for j in range(NBLKPAD // 16):
            blk = lane + j * 16
            acc = zero16
            for e in range(1, E):
                acc = acc + jnp.where(blk >= (gstart[e] >> LOG2T), 1, 0)
            bev = jnp.where(blk < used_end, acc, -1)
            bev = jnp.where(blk == NBLK, used_end, bev)
            bexp_v[pl.ds(j * 16, 16)] = bev
        pltpu.sync_copy(bexp_v, bexp_hbm)


def _ffn_kernel(be_ref, xs_ref, W1_ref, b1_ref, W2_ref, b2_ref, ys_ref):
    i = pl.program_id(0)

    @pl.when(be_ref[i] >= 0)
    def _():
        xb = xs_ref[...].astype(jnp.bfloat16)
        h = jnp.maximum(
            jnp.dot(xb, W1_ref[0].astype(jnp.bfloat16),
                    preferred_element_type=jnp.float32) + b1_ref[0], 0.0)
        ys_ref[...] = jnp.dot(h.astype(jnp.bfloat16),
                              W2_ref[0].astype(jnp.bfloat16),
                              preferred_element_type=jnp.float32) + b2_ref[0]


def _ffn(bexp, xs, W1, b1r, W2, b2r):
    def xidx(i, be):
        return jnp.minimum(i, be[NBLK] - 1)

    def widx(i, be):
        return be[jnp.minimum(i, be[NBLK] - 1)]

    grid_spec = pltpu.PrefetchScalarGridSpec(
        num_scalar_prefetch=1,
        grid=(NBLK,),
        in_specs=[
            pl.BlockSpec((T, D), lambda i, be: (xidx(i, be), 0)),
            pl.BlockSpec((1, D, H), lambda i, be: (widx(i, be), 0, 0)),
            pl.BlockSpec((1, 1, H), lambda i, be: (widx(i, be), 0, 0)),
            pl.BlockSpec((1, H, D), lambda i, be: (widx(i, be), 0, 0)),
            pl.BlockSpec((1, 1, D), lambda i, be: (widx(i, be), 0, 0)),
        ],
        out_specs=pl.BlockSpec((T, D), lambda i, be: (xidx(i, be), 0)),
    )
    return pl.pallas_call(
        _ffn_kernel,
        grid_spec=grid_spec,
        out_shape=jax.ShapeDtypeStruct((NPAD, D), jnp.float32),
    )(bexp, xs, W1, b1r, W2, b2r)


@functools.cache
def _combine_call():
    return pl.kernel(
        _combine,
        out_type=jax.ShapeDtypeStruct((B, D), jnp.float32),
        mesh=_mesh(),
        scratch_types=[
            pltpu.VMEM((2, CB // 2), jnp.int32),
            pltpu.VMEM((2, CB // 2), jnp.int32),
            pltpu.VMEM((CB,), jnp.float32),
            pltpu.VMEM((CB,), jnp.float32),
            pltpu.VMEM((CB, D), jnp.float32),
            pltpu.VMEM((CB, D), jnp.float32),
            pltpu.SemaphoreType.DMA,
            pltpu.SemaphoreType.DMA,
            pltpu.SemaphoreType.DMA,
        ],
        compiler_params=pltpu.CompilerParams(needs_layout_passes=False),
    )


def _combine(ys_hbm, pos1_hbm, pos2_hbm, w1_hbm, w2_hbm,
             out_hbm,
             p1_v, p2_v, wa_v, wb_v, r1_v, r2_v, sem1, sem2, sem3):
    wid = lax.axis_index("s") * NC + lax.axis_index("c")
    base = wid * CB
    HF = CB // 2
    pltpu.sync_copy(pos1_hbm.at[pl.ds(base, HF)], p1_v.at[0])
    pltpu.sync_copy(pos1_hbm.at[pl.ds(base + HF, HF)], p1_v.at[1])
    pltpu.sync_copy(pos2_hbm.at[pl.ds(base, HF)], p2_v.at[0])
    pltpu.sync_copy(pos2_hbm.at[pl.ds(base + HF, HF)], p2_v.at[1])
    pltpu.sync_copy(w1_hbm.at[pl.ds(base, CB)], wa_v)
    pltpu.sync_copy(w2_hbm.at[pl.ds(base, CB)], wb_v)
    ga1 = pltpu.async_copy(ys_hbm.at[p1_v.at[0]], r1_v.at[pl.ds(0, HF)], sem1)
    ga2 = pltpu.async_copy(ys_hbm.at[p2_v.at[0]], r2_v.at[pl.ds(0, HF)], sem1)
    gb1 = pltpu.async_copy(ys_hbm.at[p1_v.at[1]], r1_v.at[pl.ds(HF, HF)], sem2)
    gb2 = pltpu.async_copy(ys_hbm.at[p2_v.at[1]], r2_v.at[pl.ds(HF, HF)], sem2)

    def tbody(t, carry):
        idx = jnp.full((16,), t, jnp.int32)
        wa = plsc.load_gather(wa_v, [idx])
        wb = plsc.load_gather(wb_v, [idx])
        for c in range(D // 16):
            sl = pl.ds(c * 16, 16)
            r1_v[t, sl] = r1_v[t, sl] * wa + r2_v[t, sl] * wb
        return carry

    ga1.wait()
    ga2.wait()
    lax.fori_loop(0, HF, tbody, 0)
    st = pltpu.async_copy(r1_v.at[pl.ds(0, HF)],
                          out_hbm.at[pl.ds(base, HF)], sem3)
    gb1.wait()
    gb2.wait()
    lax.fori_loop(HF, CB, tbody, 0)
    st.wait()
    pltpu.sync_copy(r1_v.at[pl.ds(HF, HF)], out_hbm.at[pl.ds(base + HF, HF)])


def kernel(x, gate_w, gate_b, W1, b1, W2, b2):
    scores, a1, a2, w1c, w2c = _gate(x, gate_w, gate_b)
    ecat = jnp.concatenate([a1.reshape(B), a2.reshape(B)])
    xs, pos1, pos2, bexp = _dispatch_call()(ecat, x)
    ys = _ffn(bexp, xs, W1, b1.reshape(E, 1, H), W2, b2.reshape(E, 1, D))
    out = _combine_call()(ys, pos1, pos2, w1c.reshape(B), w2c.reshape(B))
    return (out, lax.stop_gradient(scores))
